# Initial kernel scaffold; baseline (speedup 1.0000x reference)
import jax
import jax.numpy as jnp
from jax.experimental import pallas as pl


def kernel(x, edge_index, mut_mask, wt_idx, mut_idx, W1, b1, W2, b2, aa_emb, pos_emb, HW1, Hb1, HW2, Hb2, HW3, Hb3):
    raise NotImplementedError("write your pallas kernel here")


# speedup vs baseline: 9.0793x; 9.0793x over previous
import functools

import jax
import jax.numpy as jnp
from jax import lax
from jax.experimental import pallas as pl
from jax.experimental.pallas import tpu as pltpu
from jax.experimental.pallas import tpu_sc as plsc

N = 10000
E = 320000
IN_DIM = 128
HID = 256
Q = 4
QD = HID // Q
NC = 2
NS = 16
B = 128
ROWS = 2560
RPS = ROWS // NS
RPH = ROWS // (NC * NS)
ACC_ROWS = 10112
RPT = ACC_ROWS // NS
RB = 2000
AA_DIM = 64


def _mesh():
    return plsc.VectorSubcoreMesh(core_axis_name="c", subcore_axis_name="s")


def _make_hist():
    @functools.partial(
        pl.kernel,
        out_type=jax.ShapeDtypeStruct((NC, ACC_ROWS, 16), jnp.float32),
        mesh=_mesh(),
        compiler_params=pltpu.CompilerParams(use_tc_tiling_on_sc=False),
        scratch_types=[
            pltpu.VMEM((RPH, B), jnp.int32),
            pltpu.VMEM((B, 16), jnp.float32),
            pltpu.VMEM((RPT, 16), jnp.float32),
            pltpu.VMEM_SHARED((ACC_ROWS, 16), jnp.float32),
        ],
    )
    def hist(c2, out, cidx, ones_v, zeros_v, acc):
        cid = lax.axis_index("c")
        sid = lax.axis_index("s")

        def fill_ones(i, _):
            ones_v[i, :] = jnp.full((16,), 1.0, jnp.float32)
            return 0
        lax.fori_loop(0, B, fill_ones, 0)

        def fill_zeros(i, _):
            zeros_v[i, :] = jnp.zeros((16,), jnp.float32)
            return 0
        lax.fori_loop(0, RPT, fill_zeros, 0)

        pltpu.sync_copy(zeros_v, acc.at[pl.ds(sid * RPT, RPT)])
        plsc.subcore_barrier()

        base = cid * (NS * RPH) + sid * RPH
        pltpu.sync_copy(c2.at[pl.ds(base, RPH)], cidx)

        def body(b, _):
            pltpu.sync_copy(ones_v, acc.at[cidx.at[b]], add=True)
            return 0
        lax.fori_loop(0, RPH, body, 0)

        plsc.subcore_barrier()
        pltpu.sync_copy(acc.at[pl.ds(sid * RPT, RPT)],
                        out.at[cid, pl.ds(sid * RPT, RPT)])

    return hist


_hist = _make_hist()


def _make_spmm():
    @functools.partial(
        pl.kernel,
        out_type=jax.ShapeDtypeStruct((Q, ACC_ROWS, QD), jnp.float32),
        mesh=_mesh(),
        compiler_params=pltpu.CompilerParams(use_tc_tiling_on_sc=False),
        scratch_types=[
            pltpu.VMEM((RPS, B), jnp.int32),
            pltpu.VMEM((RPS, B), jnp.int32),
            pltpu.VMEM((B, QD), jnp.float32),
            pltpu.VMEM((B, QD), jnp.float32),
            pltpu.VMEM((64, QD), jnp.float32),
            pltpu.VMEM_SHARED((ACC_ROWS, QD), jnp.float32),
            pltpu.SemaphoreType.DMA,
            pltpu.SemaphoreType.DMA,
        ],
    )
    def spmm(g, r4, c2, out, ridx, cidx, buf0, buf1, zb, acc, sem0, sem1):
        cid = lax.axis_index("c")
        sid = lax.axis_index("s")

        def fill_zb(i, _):
            for j in range(QD // 16):
                zb[i, pl.ds(j * 16, 16)] = jnp.zeros((16,), jnp.float32)
            return 0
        lax.fori_loop(0, 64, fill_zb, 0)

        pltpu.sync_copy(c2.at[pl.ds(sid * RPS, RPS)], cidx)
        row0 = sid * RPT

        for p in range(2):
            for k in range(RPT // 64):
                pltpu.sync_copy(zb, acc.at[pl.ds(row0 + k * 64, 64)])
            rem = RPT - (RPT // 64) * 64
            pltpu.sync_copy(zb.at[pl.ds(0, rem)],
                            acc.at[pl.ds(row0 + (RPT // 64) * 64, rem)])

            qid = cid * 2 + p
            pltpu.sync_copy(r4.at[qid, pl.ds(sid * RPS, RPS)], ridx)
            plsc.subcore_barrier()

            pltpu.async_copy(g.at[ridx.at[0]], buf0, sem0)

            def body(i, _):
                b0 = 2 * i
                b1 = 2 * i + 1
                pltpu.async_copy(g.at[ridx.at[b1]], buf1, sem1)
                pltpu.make_async_copy(g.at[ridx.at[b0]], buf0, sem0).wait()
                pltpu.sync_copy(buf0, acc.at[cidx.at[b0]], add=True)

                @pl.when(i < RPS // 2 - 1)
                def _():
                    pltpu.async_copy(g.at[ridx.at[b0 + 2]], buf0, sem0)

                pltpu.make_async_copy(g.at[ridx.at[b1]], buf1, sem1).wait()
                pltpu.sync_copy(buf1, acc.at[cidx.at[b1]], add=True)
                return 0
            lax.fori_loop(0, RPS // 2, body, 0)

            plsc.subcore_barrier()
            pltpu.sync_copy(acc.at[pl.ds(row0, RPT)],
                            out.at[qid, pl.ds(row0, RPT)])

    return spmm


_spmm = _make_spmm()


def _dis_from_cnt(cnt_blk):
    return lax.rsqrt(1.0 + cnt_blk[0, :, 0] + cnt_blk[1, :, 0])


def _mm1_body(x_ref, w_ref, cnt_ref, g_ref):
    dis = _dis_from_cnt(cnt_ref[...])
    u = jnp.dot(x_ref[...], w_ref[0], preferred_element_type=jnp.float32)
    g_ref[0] = dis[:, None] * u


def _mm1(x, w1q, cnts):
    return pl.pallas_call(
        _mm1_body,
        grid=(N // RB, Q),
        in_specs=[
            pl.BlockSpec((RB, IN_DIM), lambda i, q: (i, 0)),
            pl.BlockSpec((1, IN_DIM, QD), lambda i, q: (q, 0, 0)),
            pl.BlockSpec((NC, RB, 16), lambda i, q: (0, i, 0)),
        ],
        out_specs=pl.BlockSpec((1, RB, QD), lambda i, q: (q, i, 0)),
        out_shape=jax.ShapeDtypeStruct((Q, N, QD), jnp.float32),
    )(x, w1q, cnts)


def _hsum(s_ref, g_ref):
    return jnp.concatenate(
        [s_ref[q] + g_ref[q] for q in range(Q)], axis=1)


def _mm2_body(s_ref, g_ref, cnt_ref, b1_ref, w_ref, out_ref):
    dis = _dis_from_cnt(cnt_ref[...])
    h = _hsum(s_ref, g_ref)
    b1row = jnp.concatenate([b1_ref[q] for q in range(Q)], axis=0)
    h = jnp.maximum(dis[:, None] * h + b1row[None, :], 0.0)
    u = jnp.dot(h, w_ref[0], preferred_element_type=jnp.float32)
    out_ref[0] = dis[:, None] * u


def _mm2(s1, g1, cnts, b1q, w2q):
    return pl.pallas_call(
        _mm2_body,
        grid=(N // RB, Q),
        in_specs=[
            pl.BlockSpec((Q, RB, QD), lambda i, q: (0, i, 0)),
            pl.BlockSpec((Q, RB, QD), lambda i, q: (0, i, 0)),
            pl.BlockSpec((NC, RB, 16), lambda i, q: (0, i, 0)),
            pl.BlockSpec((Q, QD), lambda i, q: (0, 0)),
            pl.BlockSpec((1, HID, QD), lambda i, q: (q, 0, 0)),
        ],
        out_specs=pl.BlockSpec((1, RB, QD), lambda i, q: (q, i, 0)),
        out_shape=jax.ShapeDtypeStruct((Q, N, QD), jnp.float32),
    )(s1, g1, cnts, b1q, w2q)


def _zred_body(s_ref, g_ref, cnt_ref, b2_ref, m_ref, z_ref):
    i = pl.program_id(0)
    dis = _dis_from_cnt(cnt_ref[...])
    h = _hsum(s_ref, g_ref)
    b2row = jnp.concatenate([b2_ref[q] for q in range(Q)], axis=0)
    h = jnp.maximum(dis[:, None] * h + b2row[None, :], 0.0)
    zp = jnp.sum(m_ref[0, 0][:, None] * h, axis=0)

    @pl.when(i == 0)
    def _():
        z_ref[...] = jnp.zeros_like(z_ref)

    z_ref[...] += zp[None, :]


def _zred(s2, g2, cnts, b2q, mmr):
    return pl.pallas_call(
        _zred_body,
        grid=(N // RB,),
        in_specs=[
            pl.BlockSpec((Q, RB, QD), lambda i: (0, i, 0)),
            pl.BlockSpec((Q, RB, QD), lambda i: (0, i, 0)),
            pl.BlockSpec((NC, RB, 16), lambda i: (0, i, 0)),
            pl.BlockSpec((Q, QD), lambda i: (0, 0)),
            pl.BlockSpec((1, 1, RB), lambda i: (i, 0, 0)),
        ],
        out_specs=pl.BlockSpec((1, HID), lambda i: (0, 0)),
        out_shape=jax.ShapeDtypeStruct((1, HID), jnp.float32),
    )(s2, g2, cnts, b2q, mmr)


def _head_body(z_ref, m_ref, wti_ref, mti_ref, aa_ref, pe_ref,
               hw1_ref, hb1_ref, hw2_ref, hb2_ref, hw3_ref, hb3_ref, o_ref):
    mm = m_ref[0]
    p = jnp.minimum(jnp.argmax(mm).astype(jnp.int32), 511)
    wt_i = wti_ref[0, 0]
    mut_i = mti_ref[0, 0]
    rows_aa = lax.broadcasted_iota(jnp.int32, (32, AA_DIM), 0)
    aa = aa_ref[...]
    wt = jnp.sum(jnp.where(rows_aa == wt_i, aa, 0.0), axis=0)
    mut = jnp.sum(jnp.where(rows_aa == mut_i, aa, 0.0), axis=0)
    rows_pe = lax.broadcasted_iota(jnp.int32, (512, 32), 0)
    pe = jnp.sum(jnp.where(rows_pe == p, pe_ref[...], 0.0), axis=0)
    feat = jnp.concatenate([z_ref[0], wt, mut, mut - wt, pe], axis=0)[None, :]
    o = jnp.maximum(
        jnp.dot(feat, hw1_ref[...], preferred_element_type=jnp.float32)
        + hb1_ref[...], 0.0)
    o = jnp.maximum(
        jnp.dot(o, hw2_ref[...], preferred_element_type=jnp.float32)
        + hb2_ref[...], 0.0)
    o_ref[...] = (jnp.sum(o * hw3_ref[...], axis=1, keepdims=True)
                  + hb3_ref[...])


def _head(z, mm1r, wti, mti, aa_pad, pos_emb, hw1, hb1r, hw2, hb2r,
          hw3r, hb3r):
    return pl.pallas_call(
        _head_body,
        out_shape=jax.ShapeDtypeStruct((1, 1), jnp.float32),
    )(z, mm1r, wti, mti, aa_pad, pos_emb, hw1, hb1r, hw2, hb2r, hw3r, hb3r)


def kernel(x, edge_index, mut_mask, wt_idx, mut_idx, W1, b1, W2, b2,
           aa_emb, pos_emb, HW1, Hb1, HW2, Hb2, HW3, Hb3):
    r = edge_index[0]
    c = edge_index[1]
    pad = ROWS * B - E
    r_pad = jnp.concatenate([r, jnp.zeros((pad,), jnp.int32)])
    c_pad = jnp.concatenate([c, jnp.full((pad,), N, jnp.int32)])
    r4 = jnp.stack([r_pad + q * N for q in range(Q)]).reshape(Q, ROWS, B)
    c2 = c_pad.reshape(ROWS, B)

    w1q = jnp.moveaxis(W1.reshape(IN_DIM, Q, QD), 1, 0)
    w2q = jnp.moveaxis(W2.reshape(HID, Q, QD), 1, 0)
    b1q = b1.reshape(Q, QD)
    b2q = b2.reshape(Q, QD)
    mmr = mut_mask.reshape(N // RB, 1, RB)
    aa_pad = jnp.concatenate(
        [aa_emb, jnp.zeros((12, AA_DIM), jnp.float32)], axis=0)
    wti = wt_idx.reshape(1, 1)
    mti = mut_idx.reshape(1, 1)
    hb1r = Hb1.reshape(1, -1)
    hb2r = Hb2.reshape(1, -1)
    hw3r = HW3.reshape(1, -1)
    hb3r = Hb3.reshape(1, 1)

    cnts = _hist(c2)
    g1 = _mm1(x, w1q, cnts)
    s1 = _spmm(g1.reshape(Q * N, QD), r4, c2)
    g2 = _mm2(s1, g1, cnts, b1q, w2q)
    s2 = _spmm(g2.reshape(Q * N, QD), r4, c2)
    z = _zred(s2, g2, cnts, b2q, mmr)
    o = _head(z, mut_mask.reshape(1, N), wti, mti, aa_pad, pos_emb,
              HW1, hb1r, HW2, hb2r, hw3r, hb3r)
    return o[0, 0]


# speedup vs baseline: 9.6660x; 1.0646x over previous
import functools

import jax
import jax.numpy as jnp
from jax import lax
from jax.experimental import pallas as pl
from jax.experimental.pallas import tpu as pltpu
from jax.experimental.pallas import tpu_sc as plsc

N = 10000
E = 320000
IN_DIM = 128
HID = 256
Q = 4
QD = HID // Q
NC = 2
NS = 16
B = 128
ROWS = 2560
RPS = ROWS // NS
RPH = ROWS // (NC * NS)
ACC_ROWS = 10112
RPT = ACC_ROWS // NS
RB = 2000
AA_DIM = 64


def _mesh():
    return plsc.VectorSubcoreMesh(core_axis_name="c", subcore_axis_name="s")


def _make_hist():
    @functools.partial(
        pl.kernel,
        out_type=jax.ShapeDtypeStruct((NC, ACC_ROWS, 16), jnp.float32),
        mesh=_mesh(),
        compiler_params=pltpu.CompilerParams(use_tc_tiling_on_sc=False),
        scratch_types=[
            pltpu.VMEM((RPH, B), jnp.int32),
            pltpu.VMEM((B, 16), jnp.float32),
            pltpu.VMEM((RPT, 16), jnp.float32),
            pltpu.VMEM_SHARED((ACC_ROWS, 16), jnp.float32),
        ],
    )
    def hist(c2, out, cidx, ones_v, zeros_v, acc):
        cid = lax.axis_index("c")
        sid = lax.axis_index("s")

        def fill_ones(i, _):
            ones_v[i, :] = jnp.full((16,), 1.0, jnp.float32)
            return 0
        lax.fori_loop(0, B, fill_ones, 0)

        def fill_zeros(i, _):
            zeros_v[i, :] = jnp.zeros((16,), jnp.float32)
            return 0
        lax.fori_loop(0, RPT, fill_zeros, 0)

        pltpu.sync_copy(zeros_v, acc.at[pl.ds(sid * RPT, RPT)])
        plsc.subcore_barrier()

        base = cid * (NS * RPH) + sid * RPH
        pltpu.sync_copy(c2.at[pl.ds(base, RPH)], cidx)

        def body(b, _):
            pltpu.sync_copy(ones_v, acc.at[cidx.at[b]], add=True)
            return 0
        lax.fori_loop(0, RPH, body, 0)

        plsc.subcore_barrier()
        pltpu.sync_copy(acc.at[pl.ds(sid * RPT, RPT)],
                        out.at[cid, pl.ds(sid * RPT, RPT)])

    return hist


_hist = _make_hist()


def _make_spmm():
    @functools.partial(
        pl.kernel,
        out_type=jax.ShapeDtypeStruct((Q, ACC_ROWS, QD), jnp.float32),
        mesh=_mesh(),
        compiler_params=pltpu.CompilerParams(use_tc_tiling_on_sc=False),
        scratch_types=[
            pltpu.VMEM((RPS, B), jnp.int32),
            pltpu.VMEM((RPS, B), jnp.int32),
            pltpu.VMEM((4, B, QD), jnp.float32),
            pltpu.VMEM((64, QD), jnp.float32),
            pltpu.VMEM_SHARED((ACC_ROWS, QD), jnp.float32),
            pltpu.SemaphoreType.DMA((4,)),
            pltpu.SemaphoreType.DMA((4,)),
        ],
    )
    def spmm(g, r4, c2, out, ridx, cidx, ring, zb, acc, gsems, ssems):
        bufs = [ring.at[s] for s in range(4)]
        gsem = [gsems.at[s] for s in range(4)]
        ssem = [ssems.at[s] for s in range(4)]
        cid = lax.axis_index("c")
        sid = lax.axis_index("s")
        NSL = 4
        NR = RPS // NSL

        def fill_zb(i, _):
            for j in range(QD // 16):
                zb[i, pl.ds(j * 16, 16)] = jnp.zeros((16,), jnp.float32)
            return 0
        lax.fori_loop(0, 64, fill_zb, 0)

        pltpu.sync_copy(c2.at[pl.ds(sid * RPS, RPS)], cidx)
        row0 = sid * RPT

        for p in range(2):
            for k in range(RPT // 64):
                pltpu.sync_copy(zb, acc.at[pl.ds(row0 + k * 64, 64)])
            rem = RPT - (RPT // 64) * 64
            pltpu.sync_copy(zb.at[pl.ds(0, rem)],
                            acc.at[pl.ds(row0 + (RPT // 64) * 64, rem)])

            qid = cid * 2 + p
            pltpu.sync_copy(r4.at[qid, pl.ds(sid * RPS, RPS)], ridx)
            plsc.subcore_barrier()

            for s in range(NSL):
                pltpu.async_copy(g.at[ridx.at[s]], bufs[s], gsem[s])

            def round_body(i, _):
                base = i * NSL
                for s in range(NSL):
                    b = base + s
                    pltpu.make_async_copy(
                        g.at[ridx.at[b]], bufs[s], gsem[s]).wait()
                    pltpu.sync_copy(bufs[s], acc.at[cidx.at[b]], add=True)

                    @pl.when(i < NR - 1)
                    def _():
                        pltpu.async_copy(
                            g.at[ridx.at[b + NSL]], bufs[s], gsem[s])
                return 0
            lax.fori_loop(0, NR, round_body, 0)

            plsc.subcore_barrier()
            pltpu.sync_copy(acc.at[pl.ds(row0, RPT)],
                            out.at[qid, pl.ds(row0, RPT)])

    return spmm


_spmm = _make_spmm()


def _dis_from_cnt(cnt_blk):
    return lax.rsqrt(1.0 + cnt_blk[0, :, 0] + cnt_blk[1, :, 0])


def _mm1_body(x_ref, w_ref, cnt_ref, g_ref):
    dis = _dis_from_cnt(cnt_ref[...])
    u = jnp.dot(x_ref[...], w_ref[0], preferred_element_type=jnp.float32)
    g_ref[0] = dis[:, None] * u


def _mm1(x, w1q, cnts):
    return pl.pallas_call(
        _mm1_body,
        grid=(N // RB, Q),
        in_specs=[
            pl.BlockSpec((RB, IN_DIM), lambda i, q: (i, 0)),
            pl.BlockSpec((1, IN_DIM, QD), lambda i, q: (q, 0, 0)),
            pl.BlockSpec((NC, RB, 16), lambda i, q: (0, i, 0)),
        ],
        out_specs=pl.BlockSpec((1, RB, QD), lambda i, q: (q, i, 0)),
        out_shape=jax.ShapeDtypeStruct((Q, N, QD), jnp.float32),
    )(x, w1q, cnts)


def _hsum(s_ref, g_ref):
    return jnp.concatenate(
        [s_ref[q] + g_ref[q] for q in range(Q)], axis=1)


def _mm2_body(s_ref, g_ref, cnt_ref, b1_ref, w_ref, out_ref):
    dis = _dis_from_cnt(cnt_ref[...])
    h = _hsum(s_ref, g_ref)
    b1row = jnp.concatenate([b1_ref[q] for q in range(Q)], axis=0)
    h = jnp.maximum(dis[:, None] * h + b1row[None, :], 0.0)
    u = jnp.dot(h, w_ref[0], preferred_element_type=jnp.float32)
    out_ref[0] = dis[:, None] * u


def _mm2(s1, g1, cnts, b1q, w2q):
    return pl.pallas_call(
        _mm2_body,
        grid=(N // RB, Q),
        in_specs=[
            pl.BlockSpec((Q, RB, QD), lambda i, q: (0, i, 0)),
            pl.BlockSpec((Q, RB, QD), lambda i, q: (0, i, 0)),
            pl.BlockSpec((NC, RB, 16), lambda i, q: (0, i, 0)),
            pl.BlockSpec((Q, QD), lambda i, q: (0, 0)),
            pl.BlockSpec((1, HID, QD), lambda i, q: (q, 0, 0)),
        ],
        out_specs=pl.BlockSpec((1, RB, QD), lambda i, q: (q, i, 0)),
        out_shape=jax.ShapeDtypeStruct((Q, N, QD), jnp.float32),
    )(s1, g1, cnts, b1q, w2q)


def _zred_body(s_ref, g_ref, cnt_ref, b2_ref, m_ref, z_ref):
    i = pl.program_id(0)
    dis = _dis_from_cnt(cnt_ref[...])
    h = _hsum(s_ref, g_ref)
    b2row = jnp.concatenate([b2_ref[q] for q in range(Q)], axis=0)
    h = jnp.maximum(dis[:, None] * h + b2row[None, :], 0.0)
    zp = jnp.sum(m_ref[0, 0][:, None] * h, axis=0)

    @pl.when(i == 0)
    def _():
        z_ref[...] = jnp.zeros_like(z_ref)

    z_ref[...] += zp[None, :]


def _zred(s2, g2, cnts, b2q, mmr):
    return pl.pallas_call(
        _zred_body,
        grid=(N // RB,),
        in_specs=[
            pl.BlockSpec((Q, RB, QD), lambda i: (0, i, 0)),
            pl.BlockSpec((Q, RB, QD), lambda i: (0, i, 0)),
            pl.BlockSpec((NC, RB, 16), lambda i: (0, i, 0)),
            pl.BlockSpec((Q, QD), lambda i: (0, 0)),
            pl.BlockSpec((1, 1, RB), lambda i: (i, 0, 0)),
        ],
        out_specs=pl.BlockSpec((1, HID), lambda i: (0, 0)),
        out_shape=jax.ShapeDtypeStruct((1, HID), jnp.float32),
    )(s2, g2, cnts, b2q, mmr)


def _head_body(z_ref, m_ref, wti_ref, mti_ref, aa_ref, pe_ref,
               hw1_ref, hb1_ref, hw2_ref, hb2_ref, hw3_ref, hb3_ref, o_ref):
    mm = m_ref[0]
    p = jnp.minimum(jnp.argmax(mm).astype(jnp.int32), 511)
    wt_i = wti_ref[0, 0]
    mut_i = mti_ref[0, 0]
    rows_aa = lax.broadcasted_iota(jnp.int32, (32, AA_DIM), 0)
    aa = aa_ref[...]
    wt = jnp.sum(jnp.where(rows_aa == wt_i, aa, 0.0), axis=0)
    mut = jnp.sum(jnp.where(rows_aa == mut_i, aa, 0.0), axis=0)
    rows_pe = lax.broadcasted_iota(jnp.int32, (512, 32), 0)
    pe = jnp.sum(jnp.where(rows_pe == p, pe_ref[...], 0.0), axis=0)
    feat = jnp.concatenate([z_ref[0], wt, mut, mut - wt, pe], axis=0)[None, :]
    o = jnp.maximum(
        jnp.dot(feat, hw1_ref[...], preferred_element_type=jnp.float32)
        + hb1_ref[...], 0.0)
    o = jnp.maximum(
        jnp.dot(o, hw2_ref[...], preferred_element_type=jnp.float32)
        + hb2_ref[...], 0.0)
    o_ref[...] = (jnp.sum(o * hw3_ref[...], axis=1, keepdims=True)
                  + hb3_ref[...])


def _head(z, mm1r, wti, mti, aa_pad, pos_emb, hw1, hb1r, hw2, hb2r,
          hw3r, hb3r):
    return pl.pallas_call(
        _head_body,
        out_shape=jax.ShapeDtypeStruct((1, 1), jnp.float32),
    )(z, mm1r, wti, mti, aa_pad, pos_emb, hw1, hb1r, hw2, hb2r, hw3r, hb3r)


def kernel(x, edge_index, mut_mask, wt_idx, mut_idx, W1, b1, W2, b2,
           aa_emb, pos_emb, HW1, Hb1, HW2, Hb2, HW3, Hb3):
    r = edge_index[0]
    c = edge_index[1]
    pad = ROWS * B - E
    r_pad = jnp.concatenate([r, jnp.zeros((pad,), jnp.int32)])
    c_pad = jnp.concatenate([c, jnp.full((pad,), N, jnp.int32)])
    r4 = jnp.stack([r_pad + q * N for q in range(Q)]).reshape(Q, ROWS, B)
    c2 = c_pad.reshape(ROWS, B)

    w1q = jnp.moveaxis(W1.reshape(IN_DIM, Q, QD), 1, 0)
    w2q = jnp.moveaxis(W2.reshape(HID, Q, QD), 1, 0)
    b1q = b1.reshape(Q, QD)
    b2q = b2.reshape(Q, QD)
    mmr = mut_mask.reshape(N // RB, 1, RB)
    aa_pad = jnp.concatenate(
        [aa_emb, jnp.zeros((12, AA_DIM), jnp.float32)], axis=0)
    wti = wt_idx.reshape(1, 1)
    mti = mut_idx.reshape(1, 1)
    hb1r = Hb1.reshape(1, -1)
    hb2r = Hb2.reshape(1, -1)
    hw3r = HW3.reshape(1, -1)
    hb3r = Hb3.reshape(1, 1)

    cnts = _hist(c2)
    g1 = _mm1(x, w1q, cnts)
    s1 = _spmm(g1.reshape(Q * N, QD), r4, c2)
    g2 = _mm2(s1, g1, cnts, b1q, w2q)
    s2 = _spmm(g2.reshape(Q * N, QD), r4, c2)
    z = _zred(s2, g2, cnts, b2q, mmr)
    o = _head(z, mut_mask.reshape(1, N), wti, mti, aa_pad, pos_emb,
              HW1, hb1r, HW2, hb2r, hw3r, hb3r)
    return o[0, 0]
